# Initial kernel scaffold; baseline (speedup 1.0000x reference)
import jax
import jax.numpy as jnp
from jax.experimental import pallas as pl


def kernel(input, embedding):
    raise NotImplementedError("write your pallas kernel here")


# speedup vs baseline: 2.4742x; 2.4742x over previous
import functools

import jax
import jax.numpy as jnp
from jax import lax
from jax.experimental import pallas as pl
from jax.experimental.pallas import tpu as pltpu
from jax.experimental.pallas import tpu_sc as plsc


_TB = 256
_KC = 128


def _argmin_body(x_ref, emb_ref, idx_ref):
    x = x_ref[...]
    tb = x.shape[0]
    k = emb_ref.shape[0]
    xa = jnp.concatenate([x, jnp.ones((tb, 1), jnp.float32)], axis=1)
    m = jnp.full((tb, 1), jnp.inf, jnp.float32)
    idx = jnp.zeros((tb, 1), jnp.int32)
    for kc in range(k // _KC):
        e_c = emb_ref[pl.ds(kc * _KC, _KC), :]
        norms_c = jnp.sum(e_c * e_c, axis=1, keepdims=True)
        ea = jnp.concatenate([-2.0 * e_c, norms_c], axis=1)
        scores = lax.dot_general(xa, ea, (((1,), (1,)), ((), ())),
                                 precision=lax.Precision.HIGHEST,
                                 preferred_element_type=jnp.float32)
        m_c = jnp.min(scores, axis=1, keepdims=True)
        k_iota = lax.broadcasted_iota(jnp.int32, scores.shape, 1) + kc * _KC
        i_c = jnp.min(jnp.where(scores <= m_c, k_iota, jnp.int32(k)),
                      axis=1, keepdims=True)
        upd = m_c < m
        idx = jnp.where(upd, i_c, idx)
        m = jnp.where(upd, m_c, m)
    idx_ref[...] = idx


def _nearest_indices(flat, embedding):
    n, c = flat.shape
    k = embedding.shape[0]
    return pl.pallas_call(
        _argmin_body,
        grid=(n // _TB,),
        in_specs=[
            pl.BlockSpec((_TB, c), lambda i: (i, 0)),
            pl.BlockSpec((k, c), lambda i: (0, 0)),
        ],
        out_specs=pl.BlockSpec((_TB, 1), lambda i: (i, 0)),
        out_shape=jax.ShapeDtypeStruct((n, 1), jnp.int32),
    )(flat, embedding)


_LANES = 128


@functools.cache
def _make_sc_gather(n, k, dtype):
    info = plsc.get_sparse_core_info()
    nw = info.num_cores * info.num_subcores
    assert n % (8 * nw) == 0
    b_per_w = n // nw
    mesh = plsc.VectorSubcoreMesh(core_axis_name="c", subcore_axis_name="s")

    @functools.partial(
        pl.kernel,
        mesh=mesh,
        out_type=jax.ShapeDtypeStruct((n, _LANES), dtype),
        scratch_types=[
            pltpu.VMEM((b_per_w,), jnp.int32),
            pltpu.VMEM((b_per_w, _LANES), dtype),
            pltpu.SemaphoreType.DMA,
        ],
    )
    def sc_gather(idx_hbm, table_hbm, out_hbm, idx_v, rows_v, sem):
        wid = lax.axis_index("s") * info.num_cores + lax.axis_index("c")
        base = wid * b_per_w
        pltpu.sync_copy(idx_hbm.at[pl.ds(base, b_per_w)], idx_v)
        pltpu.async_copy(table_hbm.at[idx_v], rows_v, sem).wait()
        pltpu.sync_copy(rows_v, out_hbm.at[pl.ds(base, b_per_w)])

    return sc_gather


def kernel(input, embedding):
    b, w, h, c = input.shape
    k = embedding.shape[0]
    n = b * w * h
    flat = input.reshape(n, c)
    idx = _nearest_indices(flat, embedding).reshape(n)
    table = jnp.pad(embedding, ((0, 0), (0, _LANES - c)))
    quant = _make_sc_gather(n, k, embedding.dtype)(idx, table)
    return quant[:, :c].reshape(b, w, h, c)


# speedup vs baseline: 2.8838x; 1.1655x over previous
import functools

import jax
import jax.numpy as jnp
from jax import lax
from jax.experimental import pallas as pl
from jax.experimental.pallas import tpu as pltpu
from jax.experimental.pallas import tpu_sc as plsc


_TB = 256
_KC = 128


def _argmin_body(x_ref, emb_ref, idx_ref):
    x = x_ref[...]
    tb = x.shape[0]
    k = emb_ref.shape[0]
    xa = jnp.concatenate([x, jnp.ones((tb, 1), jnp.float32)], axis=1)
    m = jnp.full((1, tb), jnp.inf, jnp.float32)
    idx = jnp.zeros((1, tb), jnp.int32)
    for kc in range(k // _KC):
        e_c = emb_ref[pl.ds(kc * _KC, _KC), :]
        norms_c = jnp.sum(e_c * e_c, axis=1, keepdims=True)
        ea = jnp.concatenate([-2.0 * e_c, norms_c], axis=1)
        scores = lax.dot_general(ea, xa, (((1,), (1,)), ((), ())),
                                 precision=lax.Precision.HIGHEST,
                                 preferred_element_type=jnp.float32)
        m_c = jnp.min(scores, axis=0, keepdims=True)
        k_iota = lax.broadcasted_iota(jnp.int32, scores.shape, 0) + kc * _KC
        i_c = jnp.min(jnp.where(scores <= m_c, k_iota, jnp.int32(k)),
                      axis=0, keepdims=True)
        upd = m_c < m
        idx = jnp.where(upd, i_c, idx)
        m = jnp.where(upd, m_c, m)
    idx_ref[...] = idx[None]


def _nearest_indices(flat, embedding):
    n, c = flat.shape
    k = embedding.shape[0]
    nb = n // _TB
    idx3 = pl.pallas_call(
        _argmin_body,
        grid=(nb,),
        in_specs=[
            pl.BlockSpec((_TB, c), lambda i: (i, 0)),
            pl.BlockSpec((k, c), lambda i: (0, 0)),
        ],
        out_specs=pl.BlockSpec((1, 1, _TB), lambda i: (i, 0, 0)),
        out_shape=jax.ShapeDtypeStruct((nb, 1, _TB), jnp.int32),
    )(flat, embedding)
    return idx3


@functools.cache
def _make_sc_gather(n, k, c, dtype):
    info = plsc.get_sparse_core_info()
    nw = info.num_cores * info.num_subcores
    assert n % (8 * nw) == 0
    b_per_w = n // nw
    mesh = plsc.VectorSubcoreMesh(core_axis_name="c", subcore_axis_name="s")

    @functools.partial(
        pl.kernel,
        mesh=mesh,
        out_type=jax.ShapeDtypeStruct((n, c), dtype),
        scratch_types=[
            pltpu.VMEM((b_per_w,), jnp.int32),
            pltpu.VMEM((b_per_w, c), dtype),
            pltpu.SemaphoreType.DMA,
        ],
        compiler_params=pltpu.CompilerParams(use_tc_tiling_on_sc=False),
    )
    def sc_gather(idx_hbm, table_hbm, out_hbm, idx_v, rows_v, sem):
        wid = lax.axis_index("s") * info.num_cores + lax.axis_index("c")
        base = wid * b_per_w
        pltpu.sync_copy(idx_hbm.at[pl.ds(base, b_per_w)], idx_v)
        pltpu.async_copy(table_hbm.at[idx_v], rows_v, sem).wait()
        pltpu.sync_copy(rows_v, out_hbm.at[pl.ds(base, b_per_w)])

    return sc_gather


def kernel(input, embedding):
    b, w, h, c = input.shape
    k = embedding.shape[0]
    n = b * w * h
    flat = input.reshape(n, c)
    half = n // 2
    gather = _make_sc_gather(half, k, c, embedding.dtype)
    idx0 = _nearest_indices(flat[:half], embedding).reshape(half)
    idx1 = _nearest_indices(flat[half:], embedding).reshape(half)
    q0 = gather(idx0, embedding)
    q1 = gather(idx1, embedding)
    return jnp.concatenate([q0, q1], axis=0).reshape(b, w, h, c)


# speedup vs baseline: 3.5592x; 1.2342x over previous
import functools

import jax
import jax.numpy as jnp
from jax import lax
from jax.experimental import pallas as pl
from jax.experimental.pallas import tpu as pltpu
from jax.experimental.pallas import tpu_sc as plsc


_TB = 512
_KC = 128


def _argmin_body(x_ref, emb_ref, idx_ref, ea_ref):
    x = x_ref[...]
    tb = x.shape[0]
    k = emb_ref.shape[0]
    c = emb_ref.shape[1]

    @pl.when(pl.program_id(0) == 0)
    def _():
        e = emb_ref[...]
        norms = jnp.sum(e * e, axis=1, keepdims=True)
        ea_ref[...] = jnp.concatenate([-2.0 * e, norms], axis=1)

    xa = jnp.concatenate([x, jnp.ones((tb, 1), jnp.float32)], axis=1)
    m = jnp.full((1, tb), jnp.inf, jnp.float32)
    idx = jnp.zeros((1, tb), jnp.int32)
    for kc in range(k // _KC):
        ea = ea_ref[pl.ds(kc * _KC, _KC), :]
        scores = lax.dot_general(ea, xa, (((1,), (1,)), ((), ())),
                                 precision=lax.Precision.HIGHEST,
                                 preferred_element_type=jnp.float32)
        m_c = jnp.min(scores, axis=0, keepdims=True)
        k_iota = lax.broadcasted_iota(jnp.int32, scores.shape, 0) + kc * _KC
        i_c = jnp.min(jnp.where(scores <= m_c, k_iota, jnp.int32(k)),
                      axis=0, keepdims=True)
        upd = m_c < m
        idx = jnp.where(upd, i_c, idx)
        m = jnp.where(upd, m_c, m)
    idx_ref[...] = idx[None]


def _nearest_indices(flat, embedding):
    n, c = flat.shape
    k = embedding.shape[0]
    nb = n // _TB
    idx3 = pl.pallas_call(
        _argmin_body,
        grid=(nb,),
        in_specs=[
            pl.BlockSpec((_TB, c), lambda i: (i, 0)),
            pl.BlockSpec((k, c), lambda i: (0, 0)),
        ],
        out_specs=pl.BlockSpec((1, 1, _TB), lambda i: (i, 0, 0)),
        out_shape=jax.ShapeDtypeStruct((nb, 1, _TB), jnp.int32),
        scratch_shapes=[pltpu.VMEM((k, c + 1), jnp.float32)],
    )(flat, embedding)
    return idx3


@functools.cache
def _make_sc_gather(n, k, c, dtype):
    info = plsc.get_sparse_core_info()
    nw = info.num_cores * info.num_subcores
    assert n % (8 * nw) == 0
    b_per_w = n // nw
    mesh = plsc.VectorSubcoreMesh(core_axis_name="c", subcore_axis_name="s")

    @functools.partial(
        pl.kernel,
        mesh=mesh,
        out_type=jax.ShapeDtypeStruct((n, c), dtype),
        scratch_types=[
            pltpu.VMEM((b_per_w,), jnp.int32),
            pltpu.VMEM((b_per_w, c), dtype),
            pltpu.SemaphoreType.DMA,
        ],
        compiler_params=pltpu.CompilerParams(use_tc_tiling_on_sc=False),
    )
    def sc_gather(idx_hbm, table_hbm, out_hbm, idx_v, rows_v, sem):
        wid = lax.axis_index("s") * info.num_cores + lax.axis_index("c")
        base = wid * b_per_w
        pltpu.sync_copy(idx_hbm.at[pl.ds(base, b_per_w)], idx_v)
        pltpu.async_copy(table_hbm.at[idx_v], rows_v, sem).wait()
        pltpu.sync_copy(rows_v, out_hbm.at[pl.ds(base, b_per_w)])

    return sc_gather


def kernel(input, embedding):
    b, w, h, c = input.shape
    k = embedding.shape[0]
    n = b * w * h
    flat = input.reshape(n, c)
    idx = _nearest_indices(flat, embedding).reshape(n)
    quant = _make_sc_gather(n, k, c, embedding.dtype)(idx, embedding)
    return quant.reshape(b, w, h, c)


# speedup vs baseline: 3.6571x; 1.0275x over previous
import functools

import jax
import jax.numpy as jnp
from jax import lax
from jax.experimental import pallas as pl
from jax.experimental.pallas import tpu as pltpu
from jax.experimental.pallas import tpu_sc as plsc


_TB = 1024
_KC = 128


def _argmin_body(x_ref, emb_ref, idx_ref, ea_ref):
    x = x_ref[...]
    tb = x.shape[0]
    k = emb_ref.shape[0]
    c = emb_ref.shape[1]

    @pl.when(pl.program_id(0) == 0)
    def _():
        e = emb_ref[...]
        norms = jnp.sum(e * e, axis=1, keepdims=True)
        ea_ref[...] = jnp.concatenate([-2.0 * e, norms], axis=1)

    xa = jnp.concatenate([x, jnp.ones((tb, 1), jnp.float32)], axis=1)
    m = jnp.full((1, tb), jnp.inf, jnp.float32)
    idx = jnp.zeros((1, tb), jnp.int32)
    for kc in range(k // _KC):
        ea = ea_ref[pl.ds(kc * _KC, _KC), :]
        scores = lax.dot_general(ea, xa, (((1,), (1,)), ((), ())),
                                 precision=lax.Precision.HIGHEST,
                                 preferred_element_type=jnp.float32)
        m_c = jnp.min(scores, axis=0, keepdims=True)
        k_iota = lax.broadcasted_iota(jnp.int32, scores.shape, 0) + kc * _KC
        i_c = jnp.min(jnp.where(scores <= m_c, k_iota, jnp.int32(k)),
                      axis=0, keepdims=True)
        upd = m_c < m
        idx = jnp.where(upd, i_c, idx)
        m = jnp.where(upd, m_c, m)
    idx_ref[...] = idx[None]


def _nearest_indices(flat, embedding):
    n, c = flat.shape
    k = embedding.shape[0]
    nb = n // _TB
    idx3 = pl.pallas_call(
        _argmin_body,
        grid=(nb,),
        in_specs=[
            pl.BlockSpec((_TB, c), lambda i: (i, 0)),
            pl.BlockSpec((k, c), lambda i: (0, 0)),
        ],
        out_specs=pl.BlockSpec((1, 1, _TB), lambda i: (i, 0, 0)),
        out_shape=jax.ShapeDtypeStruct((nb, 1, _TB), jnp.int32),
        scratch_shapes=[pltpu.VMEM((k, c + 1), jnp.float32)],
    )(flat, embedding)
    return idx3


@functools.cache
def _make_sc_gather(n, k, c, dtype):
    info = plsc.get_sparse_core_info()
    nw = info.num_cores * info.num_subcores
    assert n % (8 * nw) == 0
    b_per_w = n // nw
    mesh = plsc.VectorSubcoreMesh(core_axis_name="c", subcore_axis_name="s")

    @functools.partial(
        pl.kernel,
        mesh=mesh,
        out_type=jax.ShapeDtypeStruct((n, c), dtype),
        scratch_types=[
            pltpu.VMEM((b_per_w,), jnp.int32),
            pltpu.VMEM((b_per_w, c), dtype),
            pltpu.SemaphoreType.DMA,
        ],
        compiler_params=pltpu.CompilerParams(use_tc_tiling_on_sc=False),
    )
    def sc_gather(idx_hbm, table_hbm, out_hbm, idx_v, rows_v, sem):
        wid = lax.axis_index("s") * info.num_cores + lax.axis_index("c")
        base = wid * b_per_w
        pltpu.sync_copy(idx_hbm.at[pl.ds(base, b_per_w)], idx_v)
        pltpu.async_copy(table_hbm.at[idx_v], rows_v, sem).wait()
        pltpu.sync_copy(rows_v, out_hbm.at[pl.ds(base, b_per_w)])

    return sc_gather


def kernel(input, embedding):
    b, w, h, c = input.shape
    k = embedding.shape[0]
    n = b * w * h
    flat = input.reshape(n, c)
    idx = _nearest_indices(flat, embedding).reshape(n)
    quant = _make_sc_gather(n, k, c, embedding.dtype)(idx, embedding)
    return quant.reshape(b, w, h, c)


# speedup vs baseline: 3.6688x; 1.0032x over previous
import functools

import jax
import jax.numpy as jnp
from jax import lax
from jax.experimental import pallas as pl
from jax.experimental.pallas import tpu as pltpu
from jax.experimental.pallas import tpu_sc as plsc


_TB = 4096
_KC = 128


def _argmin_body(x_ref, emb_ref, idx_ref, ea_ref):
    x = x_ref[...]
    tb = x.shape[0]
    k = emb_ref.shape[0]
    c = emb_ref.shape[1]

    @pl.when(pl.program_id(0) == 0)
    def _():
        e = emb_ref[...]
        norms = jnp.sum(e * e, axis=1, keepdims=True)
        ea_ref[...] = jnp.concatenate([-2.0 * e, norms], axis=1)

    xa = jnp.concatenate([x, jnp.ones((tb, 1), jnp.float32)], axis=1)
    m = jnp.full((1, tb), jnp.inf, jnp.float32)
    idx = jnp.zeros((1, tb), jnp.int32)
    for kc in range(k // _KC):
        ea = ea_ref[pl.ds(kc * _KC, _KC), :]
        scores = lax.dot_general(ea, xa, (((1,), (1,)), ((), ())),
                                 precision=lax.Precision.HIGHEST,
                                 preferred_element_type=jnp.float32)
        m_c = jnp.min(scores, axis=0, keepdims=True)
        k_iota = lax.broadcasted_iota(jnp.int32, scores.shape, 0) + kc * _KC
        i_c = jnp.min(jnp.where(scores <= m_c, k_iota, jnp.int32(k)),
                      axis=0, keepdims=True)
        upd = m_c < m
        idx = jnp.where(upd, i_c, idx)
        m = jnp.where(upd, m_c, m)
    idx_ref[...] = idx[None]


def _nearest_indices(flat, embedding):
    n, c = flat.shape
    k = embedding.shape[0]
    nb = n // _TB
    idx3 = pl.pallas_call(
        _argmin_body,
        grid=(nb,),
        in_specs=[
            pl.BlockSpec((_TB, c), lambda i: (i, 0)),
            pl.BlockSpec((k, c), lambda i: (0, 0)),
        ],
        out_specs=pl.BlockSpec((1, 1, _TB), lambda i: (i, 0, 0)),
        out_shape=jax.ShapeDtypeStruct((nb, 1, _TB), jnp.int32),
        scratch_shapes=[pltpu.VMEM((k, c + 1), jnp.float32)],
    )(flat, embedding)
    return idx3


@functools.cache
def _make_sc_gather(n, k, c, dtype):
    info = plsc.get_sparse_core_info()
    nw = info.num_cores * info.num_subcores
    assert n % (8 * nw) == 0
    b_per_w = n // nw
    mesh = plsc.VectorSubcoreMesh(core_axis_name="c", subcore_axis_name="s")

    @functools.partial(
        pl.kernel,
        mesh=mesh,
        out_type=jax.ShapeDtypeStruct((n, c), dtype),
        scratch_types=[
            pltpu.VMEM((b_per_w,), jnp.int32),
            pltpu.VMEM((b_per_w, c), dtype),
            pltpu.SemaphoreType.DMA,
        ],
        compiler_params=pltpu.CompilerParams(use_tc_tiling_on_sc=False),
    )
    def sc_gather(idx_hbm, table_hbm, out_hbm, idx_v, rows_v, sem):
        wid = lax.axis_index("s") * info.num_cores + lax.axis_index("c")
        base = wid * b_per_w
        pltpu.sync_copy(idx_hbm.at[pl.ds(base, b_per_w)], idx_v)
        pltpu.async_copy(table_hbm.at[idx_v], rows_v, sem).wait()
        pltpu.sync_copy(rows_v, out_hbm.at[pl.ds(base, b_per_w)])

    return sc_gather


def kernel(input, embedding):
    b, w, h, c = input.shape
    k = embedding.shape[0]
    n = b * w * h
    flat = input.reshape(n, c)
    idx = _nearest_indices(flat, embedding).reshape(n)
    quant = _make_sc_gather(n, k, c, embedding.dtype)(idx, embedding)
    return quant.reshape(b, w, h, c)


# speedup vs baseline: 3.6759x; 1.0019x over previous
import functools

import jax
import jax.numpy as jnp
from jax import lax
from jax.experimental import pallas as pl
from jax.experimental.pallas import tpu as pltpu
from jax.experimental.pallas import tpu_sc as plsc


_KC = 128


def _argmin_body(x_ref, emb_ref, idx_ref, ea_ref):
    bb, ww, hh, cc = x_ref.shape
    x = x_ref[...].reshape(bb * ww * hh, cc)
    tb = x.shape[0]
    k = emb_ref.shape[0]

    e = emb_ref[...]
    norms = jnp.sum(e * e, axis=1, keepdims=True)
    ea_ref[...] = jnp.concatenate([-2.0 * e, norms], axis=1)

    xa = jnp.concatenate([x, jnp.ones((tb, 1), jnp.float32)], axis=1)
    m = jnp.full((1, tb), jnp.inf, jnp.float32)
    idx = jnp.zeros((1, tb), jnp.int32)
    for kc in range(k // _KC):
        ea = ea_ref[pl.ds(kc * _KC, _KC), :]
        scores = lax.dot_general(ea, xa, (((1,), (1,)), ((), ())),
                                 precision=lax.Precision.HIGHEST,
                                 preferred_element_type=jnp.float32)
        m_c = jnp.min(scores, axis=0, keepdims=True)
        k_iota = lax.broadcasted_iota(jnp.int32, scores.shape, 0) + kc * _KC
        i_c = jnp.min(jnp.where(scores <= m_c, k_iota, jnp.int32(k)),
                      axis=0, keepdims=True)
        upd = m_c < m
        idx = jnp.where(upd, i_c, idx)
        m = jnp.where(upd, m_c, m)
    idx_ref[...] = idx


def _nearest_indices(input4d, embedding):
    b, w, h, c = input4d.shape
    n = b * w * h
    k = embedding.shape[0]
    return pl.pallas_call(
        _argmin_body,
        out_shape=jax.ShapeDtypeStruct((1, n), jnp.int32),
        scratch_shapes=[pltpu.VMEM((k, c + 1), jnp.float32)],
    )(input4d, embedding)


@functools.cache
def _make_sc_gather(b, w, h, c, k, dtype):
    n = b * w * h
    info = plsc.get_sparse_core_info()
    nw = info.num_cores * info.num_subcores
    assert n % (8 * nw) == 0
    b_per_w = n // nw
    wpw = b_per_w // h
    wpb = w // wpw
    mesh = plsc.VectorSubcoreMesh(core_axis_name="c", subcore_axis_name="s")

    @functools.partial(
        pl.kernel,
        mesh=mesh,
        out_type=jax.ShapeDtypeStruct((b, w, h, c), dtype),
        scratch_types=[
            pltpu.VMEM((b_per_w,), jnp.int32),
            pltpu.VMEM((wpw, h, c), dtype),
            pltpu.SemaphoreType.DMA,
        ],
        compiler_params=pltpu.CompilerParams(use_tc_tiling_on_sc=False),
    )
    def sc_gather(idx_hbm, table_hbm, out_hbm, idx_v, rows_v, sem):
        wid = lax.axis_index("s") * info.num_cores + lax.axis_index("c")
        base = wid * b_per_w
        pltpu.sync_copy(idx_hbm.at[pl.ds(base, b_per_w)], idx_v)
        copies = [
            pltpu.async_copy(
                table_hbm.at[idx_v.at[pl.ds(j * h, h)]], rows_v.at[j], sem)
            for j in range(wpw)
        ]
        for cp in copies:
            cp.wait()
        b0 = wid // wpb
        w0 = (wid % wpb) * wpw
        pltpu.sync_copy(rows_v, out_hbm.at[b0, pl.ds(w0, wpw)])

    return sc_gather


def kernel(input, embedding):
    b, w, h, c = input.shape
    k = embedding.shape[0]
    n = b * w * h
    idx = _nearest_indices(input, embedding).reshape(n)
    return _make_sc_gather(b, w, h, c, k, embedding.dtype)(idx, embedding)


# speedup vs baseline: 3.8807x; 1.0557x over previous
import functools

import jax
import jax.numpy as jnp
from jax import lax
from jax.experimental import pallas as pl
from jax.experimental.pallas import tpu as pltpu
from jax.experimental.pallas import tpu_sc as plsc


_KC = 128


def _argmin_body(x_ref, emb_ref, idx_ref, ea_ref):
    bb, ww, hh, cc = x_ref.shape
    x = x_ref[...].reshape(bb * ww * hh, cc)
    tb = x.shape[0]
    k = emb_ref.shape[0]

    e = emb_ref[...]
    norms = jnp.sum(e * e, axis=1, keepdims=True)
    ea_ref[...] = jnp.concatenate([-2.0 * e, norms], axis=1)

    xa = jnp.concatenate([x, jnp.ones((tb, 1), jnp.float32)], axis=1)
    m = jnp.full((1, tb), jnp.inf, jnp.float32)
    idx = jnp.zeros((1, tb), jnp.int32)
    for kc in range(k // _KC):
        ea = ea_ref[pl.ds(kc * _KC, _KC), :]
        scores = lax.dot_general(ea, xa, (((1,), (1,)), ((), ())),
                                 precision=lax.Precision.HIGHEST,
                                 preferred_element_type=jnp.float32)
        m_c = jnp.min(scores, axis=0, keepdims=True)
        k_iota = lax.broadcasted_iota(jnp.int32, scores.shape, 0) + kc * _KC
        i_c = jnp.min(jnp.where(scores <= m_c, k_iota, jnp.int32(k)),
                      axis=0, keepdims=True)
        upd = m_c < m
        idx = jnp.where(upd, i_c, idx)
        m = jnp.where(upd, m_c, m)
    idx_ref[...] = idx


def _nearest_indices(input4d, embedding):
    b, w, h, c = input4d.shape
    n = b * w * h
    k = embedding.shape[0]
    return pl.pallas_call(
        _argmin_body,
        out_shape=jax.ShapeDtypeStruct((1, n), jnp.int32),
        scratch_shapes=[pltpu.VMEM((k, c + 1), jnp.float32)],
    )(input4d, embedding)


@functools.cache
def _make_sc_gather(b, w, h, c, k, dtype):
    n = b * w * h
    info = plsc.get_sparse_core_info()
    nw = info.num_subcores
    assert n % (8 * nw) == 0
    b_per_w = n // nw
    wpw = b_per_w // h
    wpb = w // wpw
    mesh = plsc.VectorSubcoreMesh(core_axis_name="c", subcore_axis_name="s", num_cores=1)

    @functools.partial(
        pl.kernel,
        mesh=mesh,
        out_type=jax.ShapeDtypeStruct((b, w, h, c), dtype),
        scratch_types=[
            pltpu.VMEM((b_per_w,), jnp.int32),
            pltpu.VMEM((wpw, h, c), dtype),
            pltpu.SemaphoreType.DMA,
        ],
        compiler_params=pltpu.CompilerParams(use_tc_tiling_on_sc=False),
    )
    def sc_gather(idx_hbm, table_hbm, out_hbm, idx_v, rows_v, sem):
        wid = lax.axis_index("s")
        base = wid * b_per_w
        pltpu.sync_copy(idx_hbm.at[pl.ds(base, b_per_w)], idx_v)
        copies = [
            pltpu.async_copy(
                table_hbm.at[idx_v.at[pl.ds(j * h, h)]], rows_v.at[j], sem)
            for j in range(wpw)
        ]
        for cp in copies:
            cp.wait()
        b0 = wid // wpb
        w0 = (wid % wpb) * wpw
        pltpu.sync_copy(rows_v, out_hbm.at[b0, pl.ds(w0, wpw)])

    return sc_gather


def kernel(input, embedding):
    b, w, h, c = input.shape
    k = embedding.shape[0]
    n = b * w * h
    idx = _nearest_indices(input, embedding).reshape(n)
    return _make_sc_gather(b, w, h, c, k, embedding.dtype)(idx, embedding)
